# baseline (speedup 1.0000x reference)
import math
import functools

import jax
import jax.numpy as jnp
from jax import lax
from jax.experimental import pallas as pl
from jax.experimental.pallas import tpu as pltpu
from jax.experimental.pallas import tpu_sc as plsc

H = 64
W = 2048
HW = H * W
KNN_K = 5
SEARCH = 5
S2 = SEARCH * SEARCH
CENTER = (S2 - 1) // 2
SIGMA = 1.0
CUTOFF = 1.0
NCLASSES = 20

NW = 32
BLK = 512
NBLK = 8
NP = NW * NBLK * BLK

OOB_SLOT = HW
CUT_SLOT = HW + 1
L = 16
GROUPS = BLK // L

TQ = HW // 8
SENT_ROW = 4 * TQ


def _inv_gauss_weights():
    coords = jnp.arange(SEARCH, dtype=jnp.float32)
    x_grid = jnp.broadcast_to(coords[None, :], (SEARCH, SEARCH))
    y_grid = x_grid.T
    mean = (SEARCH - 1) / 2.0
    g = (1.0 / (2.0 * math.pi * SIGMA ** 2)) * jnp.exp(
        -((x_grid - mean) ** 2 + (y_grid - mean) ** 2) / (2.0 * SIGMA ** 2))
    g = g / jnp.sum(g)
    return (1.0 - g).reshape(-1)


def _knn_body(rng_hbm, cls_hbm, px_hbm, py_hbm, u_hbm, w_hbm, out_hbm,
              pxA, pyA, uA, idxA, grA, selA, gcA, obA,
              pxB, pyB, uB, idxB, grB, selB, gcB, obB,
              wb, spr, spc,
              sPA, sRA, sCA, sOA, sPB, sRB, sCB, sOB):
    wid = lax.axis_index("s") * 2 + lax.axis_index("c")
    tile_base = wid * (NBLK * BLK)
    pltpu.sync_copy(w_hbm, wb)

    @pl.when(lax.axis_index("s") == 0)
    def _():
        pltpu.sync_copy(rng_hbm, spr)
        pltpu.sync_copy(cls_hbm, spc)
    plsc.subcore_barrier()

    bufA = (pxA, pyA, uA, idxA, grA, selA, gcA, obA, sPA, sRA, sCA, sOA)
    bufB = (pxB, pyB, uB, idxB, grB, selB, gcB, obB, sPB, sRB, sCB, sOB)

    def pts_desc(buf, blk):
        px, py, u = buf[0], buf[1], buf[2]
        base = tile_base + blk * BLK
        return (pltpu.make_async_copy(px_hbm.at[pl.ds(base, BLK)], px, buf[8]),
                pltpu.make_async_copy(py_hbm.at[pl.ds(base, BLK)], py, buf[8]),
                pltpu.make_async_copy(u_hbm.at[pl.ds(base, BLK)], u, buf[8]))

    def issue_pts(buf, blk):
        for d in pts_desc(buf, blk):
            d.start()

    def wait_pts(buf, blk):
        for d in pts_desc(buf, blk):
            d.wait()

    def rg_desc(buf):
        return pltpu.make_async_copy(spr.at[buf[3]], buf[4], buf[9])

    def cg_desc(buf):
        return pltpu.make_async_copy(spc.at[buf[5]], buf[6], buf[10])

    def out_desc(buf, blk):
        base = tile_base + blk * BLK
        return pltpu.make_async_copy(buf[7], out_hbm.at[pl.ds(base, BLK)],
                                     buf[11])

    def idxgen(buf):
        pxb, pyb, idxt = buf[0], buf[1], buf[3]

        def gen_body(j, _):
            j16 = j * L
            py_v = pyb[pl.ds(j16, L)]
            px_v = pxb[pl.ds(j16, L)]
            colstart = (px_v - 2) & (W - 1)
            for a in range(SEARCH):
                row = py_v + (a - 2)
                oob = (row < 0) | (row >= H)
                widx = row * W + colstart
                trow = jnp.where(oob, SENT_ROW,
                                 ((widx & 6) >> 1) * TQ + (widx >> 3))
                idxt[pl.ds(a * BLK + j16, L)] = trow
            return _
        lax.fori_loop(0, GROUPS, gen_body, None)

    def select(buf):
        pxb, pyb, ub, idxt, grt, selidx = (buf[0], buf[1], buf[2], buf[3],
                                           buf[4], buf[5])

        def sel_body(j, _):
            j16 = j * L
            u_v = ub[pl.ds(j16, L)]
            py_v = pyb[pl.ds(j16, L)]
            px_v = pxb[pl.ds(j16, L)]
            lanes = jnp.arange(L, dtype=jnp.int32)
            cols = [(px_v + (b - 2)) & (W - 1) for b in range(SEARCH)]
            inf = jnp.full((L,), jnp.inf, jnp.float32)
            zero_i = jnp.zeros((L,), jnp.int32)
            NSEL = KNN_K - 1
            D = [inf] * NSEL
            I = [zero_i] * NSEL
            rows = [py_v + (a - 2) for a in range(SEARCH)]
            oobs = [(r < 0) | (r >= H) for r in rows]
            lsb = px_v & 1
            for s in range(S2):
                if s == CENTER:
                    continue
                a, b = s // SEARCH, s % SEARCH
                r_a = lanes + (a * BLK + j16)
                v = plsc.load_gather(grt, [r_a, lsb + b])
                i_v = jnp.where(oobs[a], OOB_SLOT, rows[a] * W + cols[b])
                w_v = wb[pl.ds(s * L, L)]
                d = jnp.abs(v - u_v) * w_v
                lt = [d < D[k] for k in range(NSEL)]
                nD = [jnp.where(lt[0], d, D[0])]
                nI = [jnp.where(lt[0], i_v, I[0])]
                for k in range(1, NSEL):
                    tD = jnp.where(lt[k - 1], D[k - 1], d)
                    tI = jnp.where(lt[k - 1], I[k - 1], i_v)
                    nD.append(jnp.where(lt[k], tD, D[k]))
                    nI.append(jnp.where(lt[k], tI, I[k]))
                D, I = nD, nI
            selidx[pl.ds(j16, L)] = py_v * W + px_v
            for k in range(NSEL):
                img = jnp.where(D[k] > CUTOFF, CUT_SLOT, I[k])
                selidx[pl.ds((k + 1) * BLK + j16, L)] = img
            return _
        lax.fori_loop(0, GROUPS, sel_body, None)

    def vote(buf):
        gc, outb = buf[6], buf[7]

        def vote_body(j, _):
            j16 = j * L
            c = [gc[pl.ds(k * BLK + j16, L)] for k in range(KNN_K)]
            one = jnp.ones((L,), jnp.int32)
            zero = jnp.zeros((L,), jnp.int32)
            e = {}
            for k in range(KNN_K):
                for m in range(k + 1, KNN_K):
                    e[(k, m)] = jnp.where(c[k] == c[m], one, zero)
            best = zero
            for k in range(KNN_K):
                cnt = one
                for m in range(KNN_K):
                    if m != k:
                        cnt = cnt + e[(min(k, m), max(k, m))]
                valid = (c[k] != 0) & (c[k] != NCLASSES)
                key = jnp.where(valid, cnt * 64 + (63 - c[k]), zero)
                best = jnp.maximum(best, key)
            outb[pl.ds(j16, L)] = jnp.where(best > 0, 63 - (best & 63), one)
            return _
        lax.fori_loop(0, GROUPS, vote_body, None)

    def pair_body(i, _):
        blkA = 2 * i
        issue_pts(bufA, blkA)
        wait_pts(bufA, blkA)
        idxgen(bufA)
        rg_desc(bufA).start()
        issue_pts(bufB, blkA + 1)
        wait_pts(bufB, blkA + 1)
        idxgen(bufB)
        rg_desc(bufB).start()

        rg_desc(bufA).wait()
        select(bufA)
        cg_desc(bufA).start()

        rg_desc(bufB).wait()
        select(bufB)
        cg_desc(bufB).start()

        cg_desc(bufA).wait()
        vote(bufA)
        out_desc(bufA, blkA).start()
        out_desc(bufA, blkA).wait()

        cg_desc(bufB).wait()
        vote(bufB)
        out_desc(bufB, blkA + 1).start()
        out_desc(bufB, blkA + 1).wait()
        return _

    lax.fori_loop(0, NBLK // 2, pair_body, None)


@functools.partial(jax.jit, static_argnames=())
def _knn_sc(rng_flat, cls_flat, pxp, pyp, up, wrep):
    mesh = plsc.VectorSubcoreMesh(core_axis_name="c", subcore_axis_name="s")

    def one_buf():
        return [
            pltpu.VMEM((BLK,), jnp.int32),
            pltpu.VMEM((BLK,), jnp.int32),
            pltpu.VMEM((BLK,), jnp.float32),
            pltpu.VMEM((SEARCH * BLK,), jnp.int32),
            pltpu.VMEM((SEARCH * BLK, 8), jnp.float32),
            pltpu.VMEM((KNN_K * BLK,), jnp.int32),
            pltpu.VMEM((KNN_K * BLK,), jnp.int32),
            pltpu.VMEM((BLK,), jnp.int32),
        ]

    f = pl.kernel(
        _knn_body,
        out_type=jax.ShapeDtypeStruct((NP,), jnp.int32),
        mesh=mesh,
        compiler_params=pltpu.CompilerParams(needs_layout_passes=False,
                                             use_tc_tiling_on_sc=False),
        scratch_types=(
            one_buf() + one_buf()
            + [pltpu.VMEM((S2 * L,), jnp.float32)]
            + [pltpu.VMEM_SHARED((4 * TQ + 8, 8), jnp.float32),
               pltpu.VMEM_SHARED((HW + 8,), jnp.int32)]
            + [pltpu.SemaphoreType.DMA] * 8
        ),
    )
    return f(rng_flat, cls_flat, pxp, pyp, up, wrep)


def _build_row_table(img):
    F = jnp.stack([jnp.roll(img, -j, axis=1).reshape(-1) for j in range(8)])
    G = F.reshape(8, TQ, 8)[:, :, ::2]
    T = jnp.transpose(G, (2, 1, 0)).reshape(4 * TQ, 8)
    return jnp.concatenate([T, jnp.zeros((8, 8), jnp.float32)])


def kernel(proj_range, unproj_range, proj_argmax, px, py):
    N = unproj_range.shape[0]
    pad_img_c = jnp.zeros((8,), jnp.int32).at[1].set(NCLASSES)
    rng_tab = _build_row_table(proj_range)
    cls_flat = jnp.concatenate([proj_argmax.reshape(-1).astype(jnp.int32),
                                pad_img_c])
    pad_n = NP - N
    pxp = jnp.pad(px.astype(jnp.int32), (0, pad_n))
    pyp = jnp.pad(py.astype(jnp.int32), (0, pad_n))
    up = jnp.pad(unproj_range.astype(jnp.float32), (0, pad_n))
    wrep = jnp.repeat(_inv_gauss_weights(), L)
    out = _knn_sc(rng_tab, cls_flat, pxp, pyp, up, wrep)
    return out[:N].astype(jnp.int32)


# speedup vs baseline: 1.9880x; 1.9880x over previous
import math
import functools

import jax
import jax.numpy as jnp
from jax import lax
from jax.experimental import pallas as pl
from jax.experimental.pallas import tpu as pltpu
from jax.experimental.pallas import tpu_sc as plsc

H = 64
W = 2048
HW = H * W
KNN_K = 5
SEARCH = 5
S2 = SEARCH * SEARCH
CENTER = (S2 - 1) // 2
SIGMA = 1.0
CUTOFF = 1.0
NCLASSES = 20

NW = 32
BLK = 512
NBLK = 8
NP = NW * NBLK * BLK

OOB_SLOT = HW
CUT_SLOT = HW + 1
L = 16
GROUPS = BLK // L
NS1 = S2 - 1


def _inv_gauss_weights():
    coords = jnp.arange(SEARCH, dtype=jnp.float32)
    x_grid = jnp.broadcast_to(coords[None, :], (SEARCH, SEARCH))
    y_grid = x_grid.T
    mean = (SEARCH - 1) / 2.0
    g = (1.0 / (2.0 * math.pi * SIGMA ** 2)) * jnp.exp(
        -((x_grid - mean) ** 2 + (y_grid - mean) ** 2) / (2.0 * SIGMA ** 2))
    g = g / jnp.sum(g)
    return (1.0 - g).reshape(-1)


def _knn_body(rng_hbm, cls_hbm, px_hbm, py_hbm, u_hbm, w_hbm, out_hbm,
              pxA, pyA, uA, idxA, grA, selA, gcA, obA,
              pxB, pyB, uB, idxB, grB, selB, gcB, obB,
              wb, spr,
              sPA, sRA, sCA, sOA, sPB, sRB, sCB, sOB):
    wid = lax.axis_index("s") * 2 + lax.axis_index("c")
    tile_base = wid * (NBLK * BLK)
    pltpu.sync_copy(w_hbm, wb)

    @pl.when(lax.axis_index("s") == 0)
    def _():
        pltpu.sync_copy(rng_hbm, spr)
    plsc.subcore_barrier()

    bufA = (pxA, pyA, uA, idxA, grA, selA, gcA, obA, sPA, sRA, sCA, sOA)
    bufB = (pxB, pyB, uB, idxB, grB, selB, gcB, obB, sPB, sRB, sCB, sOB)

    def pts_desc(buf, blk):
        px, py, u = buf[0], buf[1], buf[2]
        base = tile_base + blk * BLK
        return (pltpu.make_async_copy(px_hbm.at[pl.ds(base, BLK)], px, buf[8]),
                pltpu.make_async_copy(py_hbm.at[pl.ds(base, BLK)], py, buf[8]),
                pltpu.make_async_copy(u_hbm.at[pl.ds(base, BLK)], u, buf[8]))

    def issue_pts(buf, blk):
        for d in pts_desc(buf, blk):
            d.start()

    def wait_pts(buf, blk):
        for d in pts_desc(buf, blk):
            d.wait()

    def rg_desc(buf):
        return pltpu.make_async_copy(spr.at[buf[3]], buf[4], buf[9])

    def cg_desc(buf):
        return pltpu.make_async_copy(cls_hbm.at[buf[5]], buf[6], buf[10])

    def out_desc(buf, blk):
        base = tile_base + blk * BLK
        return pltpu.make_async_copy(buf[7], out_hbm.at[pl.ds(base, BLK)],
                                     buf[11])

    def idxgen(buf):
        pxb, pyb, idxb = buf[0], buf[1], buf[3]

        def gen_body(j, _):
            j16 = j * L
            py_v = pyb[pl.ds(j16, L)]
            px_v = pxb[pl.ds(j16, L)]
            cols = [(px_v + (b - 2)) & (W - 1) for b in range(SEARCH)]
            for a in range(SEARCH):
                row = py_v + (a - 2)
                oob = (row < 0) | (row >= H)
                rowmul = row * W
                for b in range(SEARCH):
                    s = a * SEARCH + b
                    if s == CENTER:
                        continue
                    slot = s if s < CENTER else s - 1
                    idx = jnp.where(oob, OOB_SLOT, rowmul + cols[b])
                    idxb[pl.ds(slot * BLK + j16, L)] = idx
            return _
        lax.fori_loop(0, GROUPS, gen_body, None)

    def select(buf):
        pxb, pyb, ub, idxb, gr, selidx = (buf[0], buf[1], buf[2], buf[3],
                                          buf[4], buf[5])

        def sel_body(j, _):
            j16 = j * L
            u_v = ub[pl.ds(j16, L)]
            py_v = pyb[pl.ds(j16, L)]
            px_v = pxb[pl.ds(j16, L)]
            inf = jnp.full((L,), jnp.inf, jnp.float32)
            zero_i = jnp.zeros((L,), jnp.int32)
            NSEL = KNN_K - 1
            D = [inf] * NSEL
            I = [zero_i] * NSEL
            for s in range(S2):
                if s == CENTER:
                    continue
                slot = s if s < CENTER else s - 1
                i_v = idxb[pl.ds(slot * BLK + j16, L)]
                v = gr[pl.ds(slot * BLK + j16, L)]
                w_v = wb[pl.ds(s * L, L)]
                d = jnp.abs(v - u_v) * w_v
                lt = [d < D[k] for k in range(NSEL)]
                nD = [jnp.where(lt[0], d, D[0])]
                nI = [jnp.where(lt[0], i_v, I[0])]
                for k in range(1, NSEL):
                    tD = jnp.where(lt[k - 1], D[k - 1], d)
                    tI = jnp.where(lt[k - 1], I[k - 1], i_v)
                    nD.append(jnp.where(lt[k], tD, D[k]))
                    nI.append(jnp.where(lt[k], tI, I[k]))
                D, I = nD, nI
            selidx[pl.ds(j16, L)] = py_v * W + px_v
            for k in range(NSEL):
                img = jnp.where(D[k] > CUTOFF, CUT_SLOT, I[k])
                selidx[pl.ds((k + 1) * BLK + j16, L)] = img
            return _
        lax.fori_loop(0, GROUPS, sel_body, None)

    def vote(buf):
        gc, outb = buf[6], buf[7]

        def vote_body(j, _):
            j16 = j * L
            c = [gc[pl.ds(k * BLK + j16, L)] for k in range(KNN_K)]
            one = jnp.ones((L,), jnp.int32)
            zero = jnp.zeros((L,), jnp.int32)
            e = {}
            for k in range(KNN_K):
                for m in range(k + 1, KNN_K):
                    e[(k, m)] = jnp.where(c[k] == c[m], one, zero)
            best = zero
            for k in range(KNN_K):
                cnt = one
                for m in range(KNN_K):
                    if m != k:
                        cnt = cnt + e[(min(k, m), max(k, m))]
                valid = (c[k] != 0) & (c[k] != NCLASSES)
                key = jnp.where(valid, cnt * 64 + (63 - c[k]), zero)
                best = jnp.maximum(best, key)
            outb[pl.ds(j16, L)] = jnp.where(best > 0, 63 - (best & 63), one)
            return _
        lax.fori_loop(0, GROUPS, vote_body, None)

    def pair_body(i, _):
        blkA = 2 * i
        issue_pts(bufA, blkA)
        wait_pts(bufA, blkA)
        idxgen(bufA)
        rg_desc(bufA).start()
        issue_pts(bufB, blkA + 1)
        wait_pts(bufB, blkA + 1)
        idxgen(bufB)
        rg_desc(bufB).start()

        rg_desc(bufA).wait()
        select(bufA)
        cg_desc(bufA).start()

        rg_desc(bufB).wait()
        select(bufB)
        cg_desc(bufB).start()

        cg_desc(bufA).wait()
        vote(bufA)
        out_desc(bufA, blkA).start()
        out_desc(bufA, blkA).wait()

        cg_desc(bufB).wait()
        vote(bufB)
        out_desc(bufB, blkA + 1).start()
        out_desc(bufB, blkA + 1).wait()
        return _

    lax.fori_loop(0, NBLK // 2, pair_body, None)


@functools.partial(jax.jit, static_argnames=())
def _knn_sc(rng_flat, cls_flat, pxp, pyp, up, wrep):
    mesh = plsc.VectorSubcoreMesh(core_axis_name="c", subcore_axis_name="s")

    def one_buf():
        return [
            pltpu.VMEM((BLK,), jnp.int32),
            pltpu.VMEM((BLK,), jnp.int32),
            pltpu.VMEM((BLK,), jnp.float32),
            pltpu.VMEM((NS1 * BLK,), jnp.int32),
            pltpu.VMEM((NS1 * BLK,), jnp.float32),
            pltpu.VMEM((KNN_K * BLK,), jnp.int32),
            pltpu.VMEM((KNN_K * BLK,), jnp.int32),
            pltpu.VMEM((BLK,), jnp.int32),
        ]

    f = pl.kernel(
        _knn_body,
        out_type=jax.ShapeDtypeStruct((NP,), jnp.int32),
        mesh=mesh,
        scratch_types=(
            one_buf() + one_buf()
            + [pltpu.VMEM((S2 * L,), jnp.float32)]
            + [pltpu.VMEM_SHARED((HW + 8,), jnp.float32)]
            + [pltpu.SemaphoreType.DMA] * 8
        ),
    )
    return f(rng_flat, cls_flat, pxp, pyp, up, wrep)


def kernel(proj_range, unproj_range, proj_argmax, px, py):
    N = unproj_range.shape[0]
    pad_img_r = jnp.zeros((8,), jnp.float32)
    pad_img_c = jnp.zeros((8,), jnp.int32).at[1].set(NCLASSES)
    rng_flat = jnp.concatenate([proj_range.reshape(-1), pad_img_r])
    cls_flat = jnp.concatenate([proj_argmax.reshape(-1).astype(jnp.int32),
                                pad_img_c])
    pad_n = NP - N
    pxp = jnp.pad(px.astype(jnp.int32), (0, pad_n))
    pyp = jnp.pad(py.astype(jnp.int32), (0, pad_n))
    up = jnp.pad(unproj_range.astype(jnp.float32), (0, pad_n))
    wrep = jnp.repeat(_inv_gauss_weights(), L)
    out = _knn_sc(rng_flat, cls_flat, pxp, pyp, up, wrep)
    return out[:N].astype(jnp.int32)


# speedup vs baseline: 2.3920x; 1.2032x over previous
import math
import functools

import jax
import jax.numpy as jnp
from jax import lax
from jax.experimental import pallas as pl
from jax.experimental.pallas import tpu as pltpu
from jax.experimental.pallas import tpu_sc as plsc

H = 64
W = 2048
HW = H * W
KNN_K = 5
SEARCH = 5
S2 = SEARCH * SEARCH
CENTER = (S2 - 1) // 2
SIGMA = 1.0
CUTOFF = 1.0
NCLASSES = 20

NW = 32
BLK = 512
NBLK = 8
NP = NW * NBLK * BLK

OOB_SLOT = HW
CUT_SLOT = HW + 1
L = 16
GROUPS = BLK // L
NS1 = S2 - 1


def _inv_gauss_weights():
    coords = jnp.arange(SEARCH, dtype=jnp.float32)
    x_grid = jnp.broadcast_to(coords[None, :], (SEARCH, SEARCH))
    y_grid = x_grid.T
    mean = (SEARCH - 1) / 2.0
    g = (1.0 / (2.0 * math.pi * SIGMA ** 2)) * jnp.exp(
        -((x_grid - mean) ** 2 + (y_grid - mean) ** 2) / (2.0 * SIGMA ** 2))
    g = g / jnp.sum(g)
    return (1.0 - g).reshape(-1)


def _knn_body(rng_hbm, cls_hbm, px_hbm, py_hbm, u_hbm, w_hbm, out_hbm,
              pxA, pyA, uA, idxA, grA, selA, gcA, obA,
              pxB, pyB, uB, idxB, grB, selB, gcB, obB,
              wb, spr, spc,
              sPA, sRA, sCA, sOA, sPB, sRB, sCB, sOB):
    wid = lax.axis_index("s") * 2 + lax.axis_index("c")
    tile_base = wid * (NBLK * BLK)
    pltpu.sync_copy(w_hbm, wb)

    @pl.when(lax.axis_index("s") == 0)
    def _():
        pltpu.sync_copy(rng_hbm, spr)
        pltpu.sync_copy(cls_hbm, spc)
    plsc.subcore_barrier()

    bufA = (pxA, pyA, uA, idxA, grA, selA, gcA, obA, sPA, sRA, sCA, sOA)
    bufB = (pxB, pyB, uB, idxB, grB, selB, gcB, obB, sPB, sRB, sCB, sOB)

    def pts_desc(buf, blk):
        px, py, u = buf[0], buf[1], buf[2]
        base = tile_base + blk * BLK
        return (pltpu.make_async_copy(px_hbm.at[pl.ds(base, BLK)], px, buf[8]),
                pltpu.make_async_copy(py_hbm.at[pl.ds(base, BLK)], py, buf[8]),
                pltpu.make_async_copy(u_hbm.at[pl.ds(base, BLK)], u, buf[8]))

    def issue_pts(buf, blk):
        for d in pts_desc(buf, blk):
            d.start()

    def wait_pts(buf, blk):
        for d in pts_desc(buf, blk):
            d.wait()

    def rg_desc(buf):
        return pltpu.make_async_copy(spr.at[buf[3]], buf[4], buf[9])

    def cg_desc(buf):
        return pltpu.make_async_copy(spc.at[buf[5]], buf[6], buf[10])

    def out_desc(buf, blk):
        base = tile_base + blk * BLK
        return pltpu.make_async_copy(buf[7], out_hbm.at[pl.ds(base, BLK)],
                                     buf[11])

    def idxgen(buf):
        pxb, pyb, idxb = buf[0], buf[1], buf[3]

        def gen_body(j, _):
            j16 = j * L
            py_v = pyb[pl.ds(j16, L)]
            px_v = pxb[pl.ds(j16, L)]
            cols = [(px_v + (b - 2)) & (W - 1) for b in range(SEARCH)]
            for a in range(SEARCH):
                row = py_v + (a - 2)
                oob = (row < 0) | (row >= H)
                rowmul = row * W
                for b in range(SEARCH):
                    s = a * SEARCH + b
                    if s == CENTER:
                        continue
                    slot = s if s < CENTER else s - 1
                    idx = jnp.where(oob, OOB_SLOT, rowmul + cols[b])
                    idxb[pl.ds(slot * BLK + j16, L)] = idx
            return _
        lax.fori_loop(0, GROUPS, gen_body, None)

    def select(buf):
        pxb, pyb, ub, idxb, gr, selidx = (buf[0], buf[1], buf[2], buf[3],
                                          buf[4], buf[5])

        def sel_body(j, _):
            j16 = j * L
            u_v = ub[pl.ds(j16, L)]
            py_v = pyb[pl.ds(j16, L)]
            px_v = pxb[pl.ds(j16, L)]
            inf = jnp.full((L,), jnp.inf, jnp.float32)
            zero_i = jnp.zeros((L,), jnp.int32)
            NSEL = KNN_K - 1
            D = [inf] * NSEL
            I = [zero_i] * NSEL
            for s in range(S2):
                if s == CENTER:
                    continue
                slot = s if s < CENTER else s - 1
                i_v = idxb[pl.ds(slot * BLK + j16, L)]
                v = gr[pl.ds(slot * BLK + j16, L)]
                w_v = wb[pl.ds(s * L, L)]
                d = jnp.abs(v - u_v) * w_v
                lt = [d < D[k] for k in range(NSEL)]
                nD = [jnp.where(lt[0], d, D[0])]
                nI = [jnp.where(lt[0], i_v, I[0])]
                for k in range(1, NSEL):
                    tD = jnp.where(lt[k - 1], D[k - 1], d)
                    tI = jnp.where(lt[k - 1], I[k - 1], i_v)
                    nD.append(jnp.where(lt[k], tD, D[k]))
                    nI.append(jnp.where(lt[k], tI, I[k]))
                D, I = nD, nI
            selidx[pl.ds(j16, L)] = py_v * W + px_v
            for k in range(NSEL):
                img = jnp.where(D[k] > CUTOFF, CUT_SLOT, I[k])
                selidx[pl.ds((k + 1) * BLK + j16, L)] = img
            return _
        lax.fori_loop(0, GROUPS, sel_body, None)

    def vote(buf):
        gc, outb = buf[6], buf[7]

        def vote_body(j, _):
            j16 = j * L
            c = [gc[pl.ds(k * BLK + j16, L)] for k in range(KNN_K)]
            one = jnp.ones((L,), jnp.int32)
            zero = jnp.zeros((L,), jnp.int32)
            e = {}
            for k in range(KNN_K):
                for m in range(k + 1, KNN_K):
                    e[(k, m)] = jnp.where(c[k] == c[m], one, zero)
            best = zero
            for k in range(KNN_K):
                cnt = one
                for m in range(KNN_K):
                    if m != k:
                        cnt = cnt + e[(min(k, m), max(k, m))]
                valid = (c[k] != 0) & (c[k] != NCLASSES)
                key = jnp.where(valid, cnt * 64 + (63 - c[k]), zero)
                best = jnp.maximum(best, key)
            outb[pl.ds(j16, L)] = jnp.where(best > 0, 63 - (best & 63), one)
            return _
        lax.fori_loop(0, GROUPS, vote_body, None)

    def pair_body(i, _):
        blkA = 2 * i
        issue_pts(bufA, blkA)
        wait_pts(bufA, blkA)
        idxgen(bufA)
        rg_desc(bufA).start()
        issue_pts(bufB, blkA + 1)
        wait_pts(bufB, blkA + 1)
        idxgen(bufB)
        rg_desc(bufB).start()

        rg_desc(bufA).wait()
        select(bufA)
        cg_desc(bufA).start()

        rg_desc(bufB).wait()
        select(bufB)
        cg_desc(bufB).start()

        cg_desc(bufA).wait()
        vote(bufA)
        out_desc(bufA, blkA).start()
        out_desc(bufA, blkA).wait()

        cg_desc(bufB).wait()
        vote(bufB)
        out_desc(bufB, blkA + 1).start()
        out_desc(bufB, blkA + 1).wait()
        return _

    lax.fori_loop(0, NBLK // 2, pair_body, None)


@functools.partial(jax.jit, static_argnames=())
def _knn_sc(rng_flat, cls_flat, pxp, pyp, up, wrep):
    mesh = plsc.VectorSubcoreMesh(core_axis_name="c", subcore_axis_name="s")

    def one_buf():
        return [
            pltpu.VMEM((BLK,), jnp.int32),
            pltpu.VMEM((BLK,), jnp.int32),
            pltpu.VMEM((BLK,), jnp.float32),
            pltpu.VMEM((NS1 * BLK,), jnp.int32),
            pltpu.VMEM((NS1 * BLK,), jnp.float32),
            pltpu.VMEM((KNN_K * BLK,), jnp.int32),
            pltpu.VMEM((KNN_K * BLK,), jnp.int32),
            pltpu.VMEM((BLK,), jnp.int32),
        ]

    f = pl.kernel(
        _knn_body,
        out_type=jax.ShapeDtypeStruct((NP,), jnp.int32),
        mesh=mesh,
        scratch_types=(
            one_buf() + one_buf()
            + [pltpu.VMEM((S2 * L,), jnp.float32)]
            + [pltpu.VMEM_SHARED((HW + 8,), jnp.float32),
               pltpu.VMEM_SHARED((HW + 8,), jnp.int32)]
            + [pltpu.SemaphoreType.DMA] * 8
        ),
    )
    return f(rng_flat, cls_flat, pxp, pyp, up, wrep)


def kernel(proj_range, unproj_range, proj_argmax, px, py):
    N = unproj_range.shape[0]
    pad_img_r = jnp.zeros((8,), jnp.float32)
    pad_img_c = jnp.zeros((8,), jnp.int32).at[1].set(NCLASSES)
    rng_flat = jnp.concatenate([proj_range.reshape(-1), pad_img_r])
    cls_flat = jnp.concatenate([proj_argmax.reshape(-1).astype(jnp.int32),
                                pad_img_c])
    pad_n = NP - N
    pxp = jnp.pad(px.astype(jnp.int32), (0, pad_n))
    pyp = jnp.pad(py.astype(jnp.int32), (0, pad_n))
    up = jnp.pad(unproj_range.astype(jnp.float32), (0, pad_n))
    wrep = jnp.repeat(_inv_gauss_weights(), L)
    out = _knn_sc(rng_flat, cls_flat, pxp, pyp, up, wrep)
    return out[:N].astype(jnp.int32)
